# baseline (speedup 1.0000x reference)
import functools

import jax
import jax.numpy as jnp
from jax import lax
from jax.experimental import pallas as pl
from jax.experimental.pallas import tpu as pltpu
from jax.experimental.pallas import tpu_sc as plsc

BATCH = 4096
SEQ = 200
D_MODEL = 64
TOTAL = BATCH * SEQ
NUM_CORES = 2
NUM_SUBCORES = 16
NW = NUM_CORES * NUM_SUBCORES
PER_W = TOTAL // NW
CHUNK = 128
NCHUNK = PER_W // CHUNK
NBUF = 3
NGROUPS = (NCHUNK + NBUF - 1) // NBUF
L = 16

_mesh = plsc.VectorSubcoreMesh(core_axis_name="c", subcore_axis_name="s")


@functools.partial(
    pl.kernel,
    mesh=_mesh,
    out_type=jax.ShapeDtypeStruct((TOTAL, D_MODEL), jnp.float32),
    scratch_types=(
        [pltpu.VMEM((PER_W,), jnp.int32),
         pltpu.VMEM((NBUF, CHUNK, 2 * D_MODEL), jnp.float32),
         pltpu.VMEM((NBUF, CHUNK, D_MODEL), jnp.float32),
         pltpu.VMEM((NBUF, CHUNK), jnp.int32),
         pltpu.VMEM((NBUF, CHUNK), jnp.int32)]
        + [pltpu.SemaphoreType.DMA] * (2 * NBUF)
    ),
    compiler_params=pltpu.CompilerParams(
        use_tc_tiling_on_sc=True, needs_layout_passes=False),
)
def _embedding_gather(idx_hbm, table_hbm, out_hbm,
                      idx_v, rows_v, out_v, pidx_v, tok_v, *sems):
    gsem = sems[:NBUF]
    ssem = sems[NBUF:]
    wid = lax.axis_index("s") * NUM_CORES + lax.axis_index("c")
    base = wid * PER_W
    pltpu.sync_copy(idx_hbm.at[pl.ds(base, PER_W)], idx_v)

    def prep_and_gather(c, k):
        for bg in range(CHUNK // L):
            tok = idx_v[pl.ds(c * CHUNK + bg * L, L)]
            pidx_v[k, pl.ds(bg * L, L)] = tok >> 1
            tok_v[k, pl.ds(bg * L, L)] = tok
        pltpu.async_copy(table_hbm.at[pidx_v.at[k]], rows_v.at[k], gsem[k])

    def wait_gather(k):
        pltpu.make_async_copy(
            table_hbm.at[pidx_v.at[k]], rows_v.at[k], gsem[k]).wait()

    def start_store(c, k):
        pltpu.async_copy(
            out_v.at[k], out_hbm.at[pl.ds(base + c * CHUNK, CHUNK)], ssem[k])

    def wait_store(k):
        pltpu.make_async_copy(
            out_v.at[k], out_hbm.at[pl.ds(base, CHUNK)], ssem[k]).wait()

    def select(k):
        for bg in range(CHUNK // L):
            colv = (tok_v[k, pl.ds(bg * L, L)] & 1) * D_MODEL
            for bl in range(L):
                b = bg * L + bl
                colb = colv[bl]
                for q in range(D_MODEL // L):
                    out_v[k, b, pl.ds(q * L, L)] = (
                        rows_v[k, b, pl.ds(colb + q * L, L)])

    prep_and_gather(0, 0)
    prep_and_gather(1, 1)

    def group(g, carry):
        for j in range(NBUF):
            c = g * NBUF + j

            @pl.when(c < NCHUNK)
            def _():
                wait_gather(j)

                @pl.when(c >= NBUF)
                def _():
                    wait_store(j)

                select(j)
                start_store(c, j)

                @pl.when(c + 2 < NCHUNK)
                def _():
                    prep_and_gather(c + 2, (j + 2) % NBUF)
        return carry

    lax.fori_loop(0, NGROUPS, group, 0)

    for j in range(NBUF):
        wait_store(j)


def kernel(token_ids, W):
    idx = token_ids.reshape(TOTAL)
    table = W.reshape(500000, 2 * D_MODEL)
    out = _embedding_gather(idx, table)
    return out.reshape(BATCH, SEQ, D_MODEL)


# speedup vs baseline: 1.0370x; 1.0370x over previous
import functools

import jax
import jax.numpy as jnp
from jax import lax
from jax.experimental import pallas as pl
from jax.experimental.pallas import tpu as pltpu
from jax.experimental.pallas import tpu_sc as plsc

BATCH = 4096
SEQ = 200
D_MODEL = 64
NUM_CORES = 2
NUM_SUBCORES = 16
NW = NUM_CORES * NUM_SUBCORES
ROWS_W = BATCH // NW
NBUF = 8
DEPTH = 6
NGROUPS = ROWS_W // NBUF

_mesh = plsc.VectorSubcoreMesh(core_axis_name="c", subcore_axis_name="s")


@functools.partial(
    pl.kernel,
    mesh=_mesh,
    out_type=jax.ShapeDtypeStruct((BATCH, SEQ, D_MODEL), jnp.float32),
    scratch_types=(
        [pltpu.VMEM((ROWS_W, SEQ), jnp.int32),
         pltpu.VMEM((NBUF, SEQ, D_MODEL), jnp.float32)]
        + [pltpu.SemaphoreType.DMA] * (2 * NBUF)
    ),
    compiler_params=pltpu.CompilerParams(use_tc_tiling_on_sc=False),
)
def _embedding_gather(idx_hbm, table_hbm, out_hbm, idx_v, rows_v, *sems):
    gsem = sems[:NBUF]
    ssem = sems[NBUF:]
    wid = lax.axis_index("s") * NUM_CORES + lax.axis_index("c")
    base = wid * ROWS_W
    pltpu.sync_copy(idx_hbm.at[pl.ds(base, ROWS_W)], idx_v)

    def start_gather(r, b):
        pltpu.async_copy(table_hbm.at[idx_v.at[r]], rows_v.at[b], gsem[b])

    def wait_gather(b):
        pltpu.make_async_copy(
            table_hbm.at[idx_v.at[0]], rows_v.at[b], gsem[b]).wait()

    def start_store(r, b):
        pltpu.async_copy(rows_v.at[b], out_hbm.at[base + r], ssem[b])

    def wait_store(b):
        pltpu.make_async_copy(
            rows_v.at[b], out_hbm.at[base], ssem[b]).wait()

    for b in range(DEPTH):
        start_gather(b, b)

    for b in range(NBUF):
        i = b
        wait_gather(b)
        start_store(i, b)
        nb = (b + DEPTH) % NBUF
        if i + DEPTH - NBUF >= 0:
            wait_store(nb)
        start_gather(i + DEPTH, nb)

    def group(g, carry):
        for b in range(NBUF):
            i = g * NBUF + b
            wait_gather(b)
            start_store(i, b)
            nb = (b + DEPTH) % NBUF
            wait_store(nb)
            start_gather(i + DEPTH, nb)
        return carry

    lax.fori_loop(1, NGROUPS - 1, group, 0)

    g = NGROUPS - 1
    for b in range(NBUF):
        i = g * NBUF + b
        wait_gather(b)
        start_store(i, b)
        if i + DEPTH < ROWS_W:
            nb = (b + DEPTH) % NBUF
            wait_store(nb)
            start_gather(i + DEPTH, nb)

    for b in range(NBUF):
        wait_store(b)


def kernel(token_ids, W):
    return _embedding_gather(token_ids.astype(jnp.int32), W)
